# baseline (speedup 1.0000x reference)
import functools

import jax
import jax.numpy as jnp
from jax import lax
from jax.experimental import pallas as pl
from jax.experimental.pallas import tpu as pltpu
from jax.experimental.pallas import tpu_sc as plsc


def _sc_gather(table1, table2, idx1, idx2):
    R1, = idx1.shape
    R2, = idx2.shape
    D = table1.shape[1]
    info = plsc.get_sparse_core_info()
    NW = info.num_cores * info.num_subcores
    r1 = R1 // NW
    r2 = R2 // NW
    CH = 104
    n_ch = r1 // CH
    assert r1 % CH == 0 and n_ch % 2 == 0 and r2 % 8 == 0

    mesh = plsc.VectorSubcoreMesh(core_axis_name="c", subcore_axis_name="s")

    @functools.partial(
        pl.kernel,
        mesh=mesh,
        out_type=[
            jax.ShapeDtypeStruct((R1, D), jnp.float32),
            jax.ShapeDtypeStruct((R2, D), jnp.float32),
        ],
        scratch_types=[
            pltpu.VMEM((CH,), jnp.int32),
            pltpu.VMEM((CH, D), jnp.float32),
            pltpu.VMEM((CH,), jnp.int32),
            pltpu.VMEM((CH, D), jnp.float32),
            pltpu.VMEM((r2,), jnp.int32),
            pltpu.VMEM((r2, D), jnp.float32),
            pltpu.SemaphoreType.DMA,
            pltpu.SemaphoreType.DMA,
        ],
    )
    def k(t1, t2, i1, i2, out1, out2, idx_a, rows_a, idx_b, rows_b,
          idx2_v, rows2_v, sem_a, sem_b):
        wid = lax.axis_index("s") * info.num_cores + lax.axis_index("c")
        base1 = wid * r1

        def start(c, i_v, r_v, sem):
            pltpu.sync_copy(i1.at[pl.ds(base1 + c * CH, CH)], i_v)
            pltpu.async_copy(t1.at[i_v], r_v, sem)

        def finish(c, i_v, r_v, sem):
            pltpu.make_async_copy(t1.at[i_v], r_v, sem).wait()
            pltpu.sync_copy(r_v, out1.at[pl.ds(base1 + c * CH, CH)])

        start(0, idx_a, rows_a, sem_a)

        def pair(p, carry):
            c0 = 2 * p
            start(c0 + 1, idx_b, rows_b, sem_b)
            finish(c0, idx_a, rows_a, sem_a)

            @pl.when(c0 + 2 < n_ch)
            def _():
                start(c0 + 2, idx_a, rows_a, sem_a)

            finish(c0 + 1, idx_b, rows_b, sem_b)
            return carry

        lax.fori_loop(0, n_ch // 2, pair, 0)

        base2 = wid * r2
        pltpu.sync_copy(i2.at[pl.ds(base2, r2)], idx2_v)
        pltpu.async_copy(t2.at[idx2_v], rows2_v, sem_a).wait()
        pltpu.sync_copy(rows2_v, out2.at[pl.ds(base2, r2)])

    return k(table1, table2, idx1, idx2)


def _tc_body(N, BB, a_ref, h_ref, o_ref, w_ref, gate_ref, v_ref, ones_ref,
             ones_ref2, seg_ref, out_ref, sup_ref, gcn_ref, col_ref):
    v = v_ref[...]
    ones = ones_ref[...]
    for b in range(BB):
        sup = jnp.dot(a_ref[b], h_ref[pl.ds(b * N, N), :],
                      preferred_element_type=jnp.float32)
        sup_ref[pl.ds(b * N, N), :] = sup.astype(jnp.bfloat16)
    gcn_ref[...] = jax.nn.relu(
        jnp.dot(sup_ref[...], w_ref[...], preferred_element_type=jnp.float32))
    for b in range(BB):
        gcn = gcn_ref[pl.ds(b * N, N), :]
        tmp = jax.nn.relu(gcn * (o_ref[pl.ds(b, 1), :] * v))
        col_ref[pl.ds(b * N, N), :] = jnp.dot(
            tmp, ones, preferred_element_type=jnp.float32)
    for b in range(BB):
        lg = col_ref[pl.ds(b * N, N), :]
        col_ref[pl.ds(b * N, N), :] = jnp.exp(lg - jnp.max(lg))
    col = col_ref[...]
    e_bcast = jnp.dot(col, ones_ref2[...],
                      preferred_element_type=jnp.float32)
    seg = seg_ref[...]
    sg_raw = jnp.dot(seg, e_bcast * gcn_ref[...],
                     preferred_element_type=jnp.float32)
    ssum = jnp.dot(seg, col, preferred_element_type=jnp.float32)
    sg = sg_raw / ssum
    gate = gate_ref[...]
    out_ref[...] = gate * o_ref[...] + (1.0 - gate) * sg


def _tc_compute(A, H2d, o, W, gate, v, N, BB, interpret=False):
    B = A.shape[0]
    D = W.shape[0]
    grid = (B // BB,)
    ones_col = jnp.ones((D, 1), jnp.float32)
    seg = (jnp.arange(BB * N) // N ==
           jnp.arange(BB)[:, None]).astype(jnp.float32)
    return pl.pallas_call(
        functools.partial(_tc_body, N, BB),
        grid=grid,
        in_specs=[
            pl.BlockSpec((BB, N, N), lambda i: (i, 0, 0)),
            pl.BlockSpec((BB * N, D), lambda i: (i, 0)),
            pl.BlockSpec((BB, D), lambda i: (i, 0)),
            pl.BlockSpec((D, D), lambda i: (0, 0)),
            pl.BlockSpec((1, D), lambda i: (0, 0)),
            pl.BlockSpec((1, D), lambda i: (0, 0)),
            pl.BlockSpec((D, 1), lambda i: (0, 0)),
            pl.BlockSpec((1, D), lambda i: (0, 0)),
            pl.BlockSpec((BB, BB * N), lambda i: (0, 0)),
        ],
        out_specs=pl.BlockSpec((BB, D), lambda i: (i, 0)),
        out_shape=jax.ShapeDtypeStruct((B, D), jnp.float32),
        scratch_shapes=[
            pltpu.VMEM((BB * N, D), jnp.bfloat16),
            pltpu.VMEM((BB * N, D), jnp.float32),
            pltpu.VMEM((BB * N, 1), jnp.float32),
        ],
        interpret=interpret,
    )(A, H2d, o, W.astype(jnp.bfloat16), gate, v, ones_col,
      jnp.ones((1, D), jnp.float32), seg)


def kernel(ent_id, adj_entity_list, A, context_ent_embed, ent_embed,
           entity_gcn_weight, gate_entity, v_ent):
    B, C = adj_entity_list.shape
    N = C + 1
    D = context_ent_embed.shape[1]
    idx_all = jnp.concatenate(
        [ent_id[:, None], adj_entity_list], axis=1).reshape(B * N)
    gate = gate_entity.reshape(1, D)
    v = v_ent.reshape(1, D)
    HB = B // 2
    gathered = []
    for h in range(2):
        sl = slice(h * HB, (h + 1) * HB)
        gathered.append(_sc_gather(
            context_ent_embed, ent_embed,
            lax.dynamic_slice_in_dim(idx_all, h * HB * N, HB * N)
            .astype(jnp.int32),
            ent_id[sl].astype(jnp.int32)))
    halves = []
    for h in range(2):
        sl = slice(h * HB, (h + 1) * HB)
        H2d, o = gathered[h]
        halves.append(_tc_compute(A[sl], H2d, o, entity_gcn_weight,
                                  gate, v, N, BB=32))
    return jnp.concatenate(halves, axis=0)


# speedup vs baseline: 1.0001x; 1.0001x over previous
import functools

import jax
import jax.numpy as jnp
from jax import lax
from jax.experimental import pallas as pl
from jax.experimental.pallas import tpu as pltpu
from jax.experimental.pallas import tpu_sc as plsc


def _sc_gather(table1, table2, idx1, idx2):
    R1, = idx1.shape
    R2, = idx2.shape
    D = table1.shape[1]
    info = plsc.get_sparse_core_info()
    NW = info.num_cores * info.num_subcores
    r1 = R1 // NW
    r2 = R2 // NW
    CH = next(c for c in range(128, 0, -8) if r1 % c == 0)
    n_ch = r1 // CH
    assert r1 % CH == 0 and r2 % 8 == 0

    mesh = plsc.VectorSubcoreMesh(core_axis_name="c", subcore_axis_name="s")

    @functools.partial(
        pl.kernel,
        mesh=mesh,
        out_type=[
            jax.ShapeDtypeStruct((R1, D), jnp.float32),
            jax.ShapeDtypeStruct((R2, D), jnp.float32),
        ],
        scratch_types=[
            pltpu.VMEM((CH,), jnp.int32),
            pltpu.VMEM((CH, D), jnp.float32),
            pltpu.VMEM((CH,), jnp.int32),
            pltpu.VMEM((CH, D), jnp.float32),
            pltpu.VMEM((r2,), jnp.int32),
            pltpu.VMEM((r2, D), jnp.float32),
            pltpu.SemaphoreType.DMA,
            pltpu.SemaphoreType.DMA,
        ],
    )
    def k(t1, t2, i1, i2, out1, out2, idx_a, rows_a, idx_b, rows_b,
          idx2_v, rows2_v, sem_a, sem_b):
        wid = lax.axis_index("s") * info.num_cores + lax.axis_index("c")
        base1 = wid * r1

        def start(c, i_v, r_v, sem):
            pltpu.sync_copy(i1.at[pl.ds(base1 + c * CH, CH)], i_v)
            pltpu.async_copy(t1.at[i_v], r_v, sem)

        def finish(c, i_v, r_v, sem):
            pltpu.make_async_copy(t1.at[i_v], r_v, sem).wait()
            pltpu.sync_copy(r_v, out1.at[pl.ds(base1 + c * CH, CH)])

        start(0, idx_a, rows_a, sem_a)

        def pair(p, carry):
            c0 = 2 * p

            @pl.when(c0 + 1 < n_ch)
            def _():
                start(c0 + 1, idx_b, rows_b, sem_b)

            finish(c0, idx_a, rows_a, sem_a)

            @pl.when(c0 + 2 < n_ch)
            def _():
                start(c0 + 2, idx_a, rows_a, sem_a)

            @pl.when(c0 + 1 < n_ch)
            def _():
                finish(c0 + 1, idx_b, rows_b, sem_b)

            return carry

        lax.fori_loop(0, (n_ch + 1) // 2, pair, 0)

        base2 = wid * r2
        pltpu.sync_copy(i2.at[pl.ds(base2, r2)], idx2_v)
        pltpu.async_copy(t2.at[idx2_v], rows2_v, sem_a).wait()
        pltpu.sync_copy(rows2_v, out2.at[pl.ds(base2, r2)])

    return k(table1, table2, idx1, idx2)


def _tc_body(N, BB, a_ref, h_ref, o_ref, w_ref, gate_ref, v_ref, ones_ref,
             ones_ref2, seg_ref, out_ref, sup_ref, gcn_ref, col_ref):
    v = v_ref[...]
    ones = ones_ref[...]
    for b in range(BB):
        sup = jnp.dot(a_ref[b], h_ref[pl.ds(b * N, N), :],
                      preferred_element_type=jnp.float32)
        sup_ref[pl.ds(b * N, N), :] = sup.astype(jnp.bfloat16)
    gcn_ref[...] = jax.nn.relu(
        jnp.dot(sup_ref[...], w_ref[...], preferred_element_type=jnp.float32))
    for b in range(BB):
        gcn = gcn_ref[pl.ds(b * N, N), :]
        tmp = jax.nn.relu(gcn * (o_ref[pl.ds(b, 1), :] * v))
        col_ref[pl.ds(b * N, N), :] = jnp.dot(
            tmp, ones, preferred_element_type=jnp.float32)
    for b in range(BB):
        lg = col_ref[pl.ds(b * N, N), :]
        col_ref[pl.ds(b * N, N), :] = jnp.exp(lg - jnp.max(lg))
    col = col_ref[...]
    e_bcast = jnp.dot(col, ones_ref2[...],
                      preferred_element_type=jnp.float32)
    seg = seg_ref[...]
    sg_raw = jnp.dot(seg, e_bcast * gcn_ref[...],
                     preferred_element_type=jnp.float32)
    ssum = jnp.dot(seg, col, preferred_element_type=jnp.float32)
    sg = sg_raw / ssum
    gate = gate_ref[...]
    out_ref[...] = gate * o_ref[...] + (1.0 - gate) * sg


def _tc_compute(A, H2d, o, W, gate, v, N, BB, interpret=False):
    B = A.shape[0]
    D = W.shape[0]
    grid = (B // BB,)
    ones_col = jnp.ones((D, 1), jnp.float32)
    seg = (jnp.arange(BB * N) // N ==
           jnp.arange(BB)[:, None]).astype(jnp.float32)
    return pl.pallas_call(
        functools.partial(_tc_body, N, BB),
        grid=grid,
        in_specs=[
            pl.BlockSpec((BB, N, N), lambda i: (i, 0, 0)),
            pl.BlockSpec((BB * N, D), lambda i: (i, 0)),
            pl.BlockSpec((BB, D), lambda i: (i, 0)),
            pl.BlockSpec((D, D), lambda i: (0, 0)),
            pl.BlockSpec((1, D), lambda i: (0, 0)),
            pl.BlockSpec((1, D), lambda i: (0, 0)),
            pl.BlockSpec((D, 1), lambda i: (0, 0)),
            pl.BlockSpec((1, D), lambda i: (0, 0)),
            pl.BlockSpec((BB, BB * N), lambda i: (0, 0)),
        ],
        out_specs=pl.BlockSpec((BB, D), lambda i: (i, 0)),
        out_shape=jax.ShapeDtypeStruct((B, D), jnp.float32),
        scratch_shapes=[
            pltpu.VMEM((BB * N, D), jnp.bfloat16),
            pltpu.VMEM((BB * N, D), jnp.float32),
            pltpu.VMEM((BB * N, 1), jnp.float32),
        ],
        interpret=interpret,
    )(A, H2d, o, W.astype(jnp.bfloat16), gate, v, ones_col,
      jnp.ones((1, D), jnp.float32), seg)


def kernel(ent_id, adj_entity_list, A, context_ent_embed, ent_embed,
           entity_gcn_weight, gate_entity, v_ent):
    B, C = adj_entity_list.shape
    N = C + 1
    D = context_ent_embed.shape[1]
    idx_all = jnp.concatenate(
        [ent_id[:, None], adj_entity_list], axis=1).reshape(B * N)
    gate = gate_entity.reshape(1, D)
    v = v_ent.reshape(1, D)
    SPLIT = 4
    HB = B // SPLIT
    gathered = []
    for h in range(SPLIT):
        sl = slice(h * HB, (h + 1) * HB)
        gathered.append(_sc_gather(
            context_ent_embed, ent_embed,
            lax.dynamic_slice_in_dim(idx_all, h * HB * N, HB * N)
            .astype(jnp.int32),
            ent_id[sl].astype(jnp.int32)))
    parts = []
    for h in range(SPLIT):
        sl = slice(h * HB, (h + 1) * HB)
        H2d, o = gathered[h]
        parts.append(_tc_compute(A[sl], H2d, o, entity_gcn_weight,
                                 gate, v, N, BB=16))
    return jnp.concatenate(parts, axis=0)


# speedup vs baseline: 1.0121x; 1.0119x over previous
import functools

import jax
import jax.numpy as jnp
from jax import lax
from jax.experimental import pallas as pl
from jax.experimental.pallas import tpu as pltpu
from jax.experimental.pallas import tpu_sc as plsc


def _sc_gather(table1, table2, idx1, idx2):
    R1, = idx1.shape
    R2, = idx2.shape
    D = table1.shape[1]
    info = plsc.get_sparse_core_info()
    NW = info.num_cores * info.num_subcores
    r1 = R1 // NW
    r2 = R2 // NW
    CH = next(c for c in range(128, 0, -8) if r1 % c == 0)
    n_ch = r1 // CH
    assert r1 % CH == 0 and r2 % 8 == 0

    mesh = plsc.VectorSubcoreMesh(core_axis_name="c", subcore_axis_name="s")

    @functools.partial(
        pl.kernel,
        mesh=mesh,
        out_type=[
            jax.ShapeDtypeStruct((R1, D), jnp.float32),
            jax.ShapeDtypeStruct((R2, D), jnp.float32),
        ],
        scratch_types=[
            pltpu.VMEM((CH,), jnp.int32),
            pltpu.VMEM((CH, D), jnp.float32),
            pltpu.VMEM((CH,), jnp.int32),
            pltpu.VMEM((CH, D), jnp.float32),
            pltpu.VMEM((r2,), jnp.int32),
            pltpu.VMEM((r2, D), jnp.float32),
            pltpu.SemaphoreType.DMA,
            pltpu.SemaphoreType.DMA,
        ],
    )
    def k(t1, t2, i1, i2, out1, out2, idx_a, rows_a, idx_b, rows_b,
          idx2_v, rows2_v, sem_a, sem_b):
        wid = lax.axis_index("s") * info.num_cores + lax.axis_index("c")
        base1 = wid * r1

        def start(c, i_v, r_v, sem):
            pltpu.sync_copy(i1.at[pl.ds(base1 + c * CH, CH)], i_v)
            pltpu.async_copy(t1.at[i_v], r_v, sem)

        def finish(c, i_v, r_v, sem):
            pltpu.make_async_copy(t1.at[i_v], r_v, sem).wait()
            pltpu.sync_copy(r_v, out1.at[pl.ds(base1 + c * CH, CH)])

        start(0, idx_a, rows_a, sem_a)

        def pair(p, carry):
            c0 = 2 * p

            @pl.when(c0 + 1 < n_ch)
            def _():
                start(c0 + 1, idx_b, rows_b, sem_b)

            finish(c0, idx_a, rows_a, sem_a)

            @pl.when(c0 + 2 < n_ch)
            def _():
                start(c0 + 2, idx_a, rows_a, sem_a)

            @pl.when(c0 + 1 < n_ch)
            def _():
                finish(c0 + 1, idx_b, rows_b, sem_b)

            return carry

        lax.fori_loop(0, (n_ch + 1) // 2, pair, 0)

        base2 = wid * r2
        pltpu.sync_copy(i2.at[pl.ds(base2, r2)], idx2_v)
        pltpu.async_copy(t2.at[idx2_v], rows2_v, sem_a).wait()
        pltpu.sync_copy(rows2_v, out2.at[pl.ds(base2, r2)])

    return k(table1, table2, idx1, idx2)


def _tc_body(N, BB, a_ref, h_ref, o_ref, w_ref, gate_ref, v_ref, ones_ref,
             ones_ref2, seg_ref, out_ref, sup_ref, gcn_ref, col_ref):
    v = v_ref[...]
    ones = ones_ref[...]
    for b in range(BB):
        sup = jnp.dot(a_ref[b], h_ref[pl.ds(b * N, N), :],
                      preferred_element_type=jnp.float32)
        sup_ref[pl.ds(b * N, N), :] = sup.astype(jnp.bfloat16)
    gcn_ref[...] = jax.nn.relu(
        jnp.dot(sup_ref[...], w_ref[...], preferred_element_type=jnp.float32))
    for b in range(BB):
        gcn = gcn_ref[pl.ds(b * N, N), :]
        tmp = jax.nn.relu(gcn * (o_ref[pl.ds(b, 1), :] * v))
        col_ref[pl.ds(b * N, N), :] = jnp.dot(
            tmp, ones, preferred_element_type=jnp.float32)
    for b in range(BB):
        lg = col_ref[pl.ds(b * N, N), :]
        col_ref[pl.ds(b * N, N), :] = jnp.exp(lg - jnp.max(lg))
    col = col_ref[...]
    e_bcast = jnp.dot(col, ones_ref2[...],
                      preferred_element_type=jnp.float32)
    seg = seg_ref[...]
    sg_raw = jnp.dot(seg, e_bcast * gcn_ref[...],
                     preferred_element_type=jnp.float32)
    ssum = jnp.dot(seg, col, preferred_element_type=jnp.float32)
    sg = sg_raw / ssum
    gate = gate_ref[...]
    out_ref[...] = gate * o_ref[...] + (1.0 - gate) * sg


def _tc_compute(A, H2d, o, W, gate, v, N, BB, interpret=False):
    B = A.shape[0]
    D = W.shape[0]
    grid = (B // BB,)
    ones_col = jnp.ones((D, 1), jnp.float32)
    seg = (jnp.arange(BB * N) // N ==
           jnp.arange(BB)[:, None]).astype(jnp.float32)
    return pl.pallas_call(
        functools.partial(_tc_body, N, BB),
        grid=grid,
        in_specs=[
            pl.BlockSpec((BB, N, N), lambda i: (i, 0, 0)),
            pl.BlockSpec((BB * N, D), lambda i: (i, 0)),
            pl.BlockSpec((BB, D), lambda i: (i, 0)),
            pl.BlockSpec((D, D), lambda i: (0, 0)),
            pl.BlockSpec((1, D), lambda i: (0, 0)),
            pl.BlockSpec((1, D), lambda i: (0, 0)),
            pl.BlockSpec((D, 1), lambda i: (0, 0)),
            pl.BlockSpec((1, D), lambda i: (0, 0)),
            pl.BlockSpec((BB, BB * N), lambda i: (0, 0)),
        ],
        out_specs=pl.BlockSpec((BB, D), lambda i: (i, 0)),
        out_shape=jax.ShapeDtypeStruct((B, D), jnp.float32),
        scratch_shapes=[
            pltpu.VMEM((BB * N, D), jnp.bfloat16),
            pltpu.VMEM((BB * N, D), jnp.float32),
            pltpu.VMEM((BB * N, 1), jnp.float32),
        ],
        interpret=interpret,
    )(A, H2d, o, W.astype(jnp.bfloat16), gate, v, ones_col,
      jnp.ones((1, D), jnp.float32), seg)


def kernel(ent_id, adj_entity_list, A, context_ent_embed, ent_embed,
           entity_gcn_weight, gate_entity, v_ent):
    B, C = adj_entity_list.shape
    N = C + 1
    D = context_ent_embed.shape[1]
    idx_all = jnp.concatenate(
        [ent_id[:, None], adj_entity_list], axis=1).reshape(B * N)
    gate = gate_entity.reshape(1, D)
    v = v_ent.reshape(1, D)
    SPLIT = 2
    HB = B // SPLIT
    gathered = []
    for h in range(SPLIT):
        sl = slice(h * HB, (h + 1) * HB)
        gathered.append(_sc_gather(
            context_ent_embed, ent_embed,
            lax.dynamic_slice_in_dim(idx_all, h * HB * N, HB * N)
            .astype(jnp.int32),
            ent_id[sl].astype(jnp.int32)))
    parts = []
    for h in range(SPLIT):
        sl = slice(h * HB, (h + 1) * HB)
        H2d, o = gathered[h]
        parts.append(_tc_compute(A[sl], H2d, o, entity_gcn_weight,
                                 gate, v, N, BB=16))
    return jnp.concatenate(parts, axis=0)


# speedup vs baseline: 1.0129x; 1.0008x over previous
import functools

import jax
import jax.numpy as jnp
from jax import lax
from jax.experimental import pallas as pl
from jax.experimental.pallas import tpu as pltpu
from jax.experimental.pallas import tpu_sc as plsc


def _sc_gather(table1, table2, idx1, idx2):
    R1, = idx1.shape
    R2, = idx2.shape
    D = table1.shape[1]
    info = plsc.get_sparse_core_info()
    NW = info.num_cores * info.num_subcores
    r1 = R1 // NW
    r2 = R2 // NW
    CH = next(c for c in range(128, 0, -8) if r1 % c == 0)
    n_ch = r1 // CH
    assert r1 % CH == 0 and r2 % 8 == 0

    mesh = plsc.VectorSubcoreMesh(core_axis_name="c", subcore_axis_name="s")

    @functools.partial(
        pl.kernel,
        mesh=mesh,
        out_type=[
            jax.ShapeDtypeStruct((R1, D), jnp.float32),
            jax.ShapeDtypeStruct((R2, D), jnp.float32),
        ],
        scratch_types=[
            pltpu.VMEM((CH,), jnp.int32),
            pltpu.VMEM((CH, D), jnp.float32),
            pltpu.VMEM((CH,), jnp.int32),
            pltpu.VMEM((CH, D), jnp.float32),
            pltpu.VMEM((r2,), jnp.int32),
            pltpu.VMEM((r2, D), jnp.float32),
            pltpu.SemaphoreType.DMA,
            pltpu.SemaphoreType.DMA,
        ],
    )
    def k(t1, t2, i1, i2, out1, out2, idx_a, rows_a, idx_b, rows_b,
          idx2_v, rows2_v, sem_a, sem_b):
        wid = lax.axis_index("s") * info.num_cores + lax.axis_index("c")
        base1 = wid * r1

        def start(c, i_v, r_v, sem):
            pltpu.sync_copy(i1.at[pl.ds(base1 + c * CH, CH)], i_v)
            pltpu.async_copy(t1.at[i_v], r_v, sem)

        def finish(c, i_v, r_v, sem):
            pltpu.make_async_copy(t1.at[i_v], r_v, sem).wait()
            pltpu.sync_copy(r_v, out1.at[pl.ds(base1 + c * CH, CH)])

        start(0, idx_a, rows_a, sem_a)

        def pair(p, carry):
            c0 = 2 * p

            @pl.when(c0 + 1 < n_ch)
            def _():
                start(c0 + 1, idx_b, rows_b, sem_b)

            finish(c0, idx_a, rows_a, sem_a)

            @pl.when(c0 + 2 < n_ch)
            def _():
                start(c0 + 2, idx_a, rows_a, sem_a)

            @pl.when(c0 + 1 < n_ch)
            def _():
                finish(c0 + 1, idx_b, rows_b, sem_b)

            return carry

        lax.fori_loop(0, (n_ch + 1) // 2, pair, 0)

        base2 = wid * r2
        pltpu.sync_copy(i2.at[pl.ds(base2, r2)], idx2_v)
        pltpu.async_copy(t2.at[idx2_v], rows2_v, sem_a).wait()
        pltpu.sync_copy(rows2_v, out2.at[pl.ds(base2, r2)])

    return k(table1, table2, idx1, idx2)


def _tc_body(N, BB, a_ref, h_lo_ref, h_hi_ref, o_ref, w_ref, gate_ref, v_ref,
             ones_ref, ones_ref2, seg_ref, out_ref, sup_ref, gcn_ref, col_ref):
    v = v_ref[...]
    ones = ones_ref[...]
    HH = BB // 2
    for b in range(BB):
        h_ref = h_lo_ref if b < HH else h_hi_ref
        sup = jnp.dot(a_ref[b], h_ref[pl.ds((b % HH) * N, N), :],
                      preferred_element_type=jnp.float32)
        sup_ref[pl.ds(b * N, N), :] = sup.astype(jnp.bfloat16)
    gcn_ref[...] = jax.nn.relu(
        jnp.dot(sup_ref[...], w_ref[...], preferred_element_type=jnp.float32))
    for b in range(BB):
        gcn = gcn_ref[pl.ds(b * N, N), :]
        tmp = jax.nn.relu(gcn * (o_ref[pl.ds(b, 1), :] * v))
        col_ref[pl.ds(b * N, N), :] = jnp.dot(
            tmp, ones, preferred_element_type=jnp.float32)
    for b in range(BB):
        lg = col_ref[pl.ds(b * N, N), :]
        col_ref[pl.ds(b * N, N), :] = jnp.exp(lg - jnp.max(lg))
    col = col_ref[...]
    e_bcast = jnp.dot(col, ones_ref2[...],
                      preferred_element_type=jnp.float32)
    seg = seg_ref[...]
    sg_raw = jnp.dot(seg, e_bcast * gcn_ref[...],
                     preferred_element_type=jnp.float32)
    ssum = jnp.dot(seg, col, preferred_element_type=jnp.float32)
    sg = sg_raw / ssum
    gate = gate_ref[...]
    out_ref[...] = gate * o_ref[...] + (1.0 - gate) * sg


def _tc_compute(A, H2d, o, W, gate, v, N, BB, interpret=False):
    B = A.shape[0]
    D = W.shape[0]
    grid = (B // BB,)
    ones_col = jnp.ones((D, 1), jnp.float32)
    seg = (jnp.arange(BB * N) // N ==
           jnp.arange(BB)[:, None]).astype(jnp.float32)
    return pl.pallas_call(
        functools.partial(_tc_body, N, BB),
        grid=grid,
        in_specs=[
            pl.BlockSpec((BB, N, N), lambda i: (i, 0, 0)),
            pl.BlockSpec((BB * N // 2, D), lambda i: (2 * i, 0)),
            pl.BlockSpec((BB * N // 2, D), lambda i: (2 * i + 1, 0)),
            pl.BlockSpec((BB, D), lambda i: (i, 0)),
            pl.BlockSpec((D, D), lambda i: (0, 0)),
            pl.BlockSpec((1, D), lambda i: (0, 0)),
            pl.BlockSpec((1, D), lambda i: (0, 0)),
            pl.BlockSpec((D, 1), lambda i: (0, 0)),
            pl.BlockSpec((1, D), lambda i: (0, 0)),
            pl.BlockSpec((BB, BB * N), lambda i: (0, 0)),
        ],
        out_specs=pl.BlockSpec((BB, D), lambda i: (i, 0)),
        out_shape=jax.ShapeDtypeStruct((B, D), jnp.float32),
        scratch_shapes=[
            pltpu.VMEM((BB * N, D), jnp.bfloat16),
            pltpu.VMEM((BB * N, D), jnp.float32),
            pltpu.VMEM((BB * N, 1), jnp.float32),
        ],
        interpret=interpret,
    )(A, H2d, H2d, o, W.astype(jnp.bfloat16), gate, v, ones_col,
      jnp.ones((1, D), jnp.float32), seg)


def kernel(ent_id, adj_entity_list, A, context_ent_embed, ent_embed,
           entity_gcn_weight, gate_entity, v_ent):
    B, C = adj_entity_list.shape
    N = C + 1
    D = context_ent_embed.shape[1]
    idx_all = jnp.concatenate(
        [ent_id[:, None], adj_entity_list], axis=1).reshape(B * N)
    gate = gate_entity.reshape(1, D)
    v = v_ent.reshape(1, D)
    SPLIT = 2
    HB = B // SPLIT
    gathered = []
    for h in range(SPLIT):
        sl = slice(h * HB, (h + 1) * HB)
        gathered.append(_sc_gather(
            context_ent_embed, ent_embed,
            lax.dynamic_slice_in_dim(idx_all, h * HB * N, HB * N)
            .astype(jnp.int32),
            ent_id[sl].astype(jnp.int32)))
    parts = []
    for h in range(SPLIT):
        sl = slice(h * HB, (h + 1) * HB)
        H2d, o = gathered[h]
        parts.append(_tc_compute(A[sl], H2d, o, entity_gcn_weight,
                                 gate, v, N, BB=16))
    return jnp.concatenate(parts, axis=0)
